# baseline (speedup 1.0000x reference)
import functools

import jax
import jax.numpy as jnp
from jax import lax
from jax.experimental import pallas as pl
from jax.experimental.pallas import tpu as pltpu
from jax.experimental.pallas import tpu_sc as plsc

NUM_EXPERTS = 64
TOP_K = 2
HIDDEN = 1024
D_FF = 512
TOKENS = 128

E_BLK = 2
PRE_EXPERTS = 8
PAD_COLS = 16

_SC_INFO = plsc.get_sparse_core_info()
_NC = _SC_INFO.num_cores
_NS = _SC_INFO.num_subcores
_NW = _NC * _NS
_TPW = TOKENS // _NW


def _routing_sc_body(logits_hbm, ids_hbm, vals_hbm, loc_v, ids_v, vals_v):
    wid = lax.axis_index("s") * _NC + lax.axis_index("c")
    base = wid * _TPW
    pltpu.sync_copy(logits_hbm.at[pl.ds(base, _TPW)], loc_v)
    iota = lax.iota(jnp.int32, 16)
    for j in range(_TPW):
        rows = [loc_v[j, pl.ds(k * 16, 16)] for k in range(NUM_EXPERTS // 16)]
        m1 = jnp.max(rows[0])
        for r in rows[1:]:
            m1 = jnp.maximum(m1, jnp.max(r))
        a1 = jnp.int32(NUM_EXPERTS)
        for k, r in enumerate(rows):
            cand = jnp.min(jnp.where(r == m1, iota + 16 * k, NUM_EXPERTS))
            a1 = jnp.minimum(a1, cand)
        m2 = jnp.float32(-1e30)
        masked = []
        for k, r in enumerate(rows):
            rm = jnp.where(iota + 16 * k == a1, -1e30, r)
            masked.append(rm)
            m2 = jnp.maximum(m2, jnp.max(rm))
        a2 = jnp.int32(NUM_EXPERTS)
        for k, rm in enumerate(masked):
            cand = jnp.min(jnp.where(rm == m2, iota + 16 * k, NUM_EXPERTS))
            a2 = jnp.minimum(a2, cand)
        ids_v[j, pl.ds(0, 16)] = jnp.where(
            iota == 0, a1, jnp.where(iota == 1, a2, 0))
        vals_v[j, pl.ds(0, 16)] = jnp.where(
            iota == 0, m1, jnp.where(iota == 1, m2, 0.0))
    pltpu.sync_copy(ids_v, ids_hbm.at[pl.ds(base, _TPW)])
    pltpu.sync_copy(vals_v, vals_hbm.at[pl.ds(base, _TPW)])


_routing_sc = functools.partial(
    pl.kernel,
    out_type=[
        jax.ShapeDtypeStruct((TOKENS, PAD_COLS), jnp.int32),
        jax.ShapeDtypeStruct((TOKENS, PAD_COLS), jnp.float32),
    ],
    mesh=plsc.VectorSubcoreMesh(core_axis_name="c", subcore_axis_name="s"),
    compiler_params=pltpu.CompilerParams(needs_layout_passes=False),
    scratch_types=[
        pltpu.VMEM((_TPW, NUM_EXPERTS), jnp.float32),
        pltpu.VMEM((_TPW, PAD_COLS), jnp.int32),
        pltpu.VMEM((_TPW, PAD_COLS), jnp.float32),
    ],
)(_routing_sc_body)


def _swiglu(x, wg, wu, wd):
    gate = jnp.dot(x, wg, preferred_element_type=jnp.float32)
    up = jnp.dot(x, wu, preferred_element_type=jnp.float32)
    act = gate * jax.nn.sigmoid(gate) * up
    return jnp.dot(act, wd, preferred_element_type=jnp.float32)


def _moe_pre_kernel(x_ref, logits_ref, wg_ref, wu_ref, wd_ref, out_ref,
                    w1_ref, w2_ref, a1_ref, a2_ref):
    g = pl.program_id(0)

    @pl.when(g == 0)
    def _routing():
        logits = logits_ref[...]
        m1 = jnp.max(logits, axis=1, keepdims=True)
        a1 = jnp.argmax(logits, axis=1).reshape(TOKENS, 1)
        cols = jax.lax.broadcasted_iota(jnp.int32, (TOKENS, NUM_EXPERTS), 1)
        masked = jnp.where(cols == a1, -jnp.inf, logits)
        m2 = jnp.max(masked, axis=1, keepdims=True)
        a2 = jnp.argmax(masked, axis=1).reshape(TOKENS, 1)
        e2 = jnp.exp(m2 - m1)
        w1_ref[...] = 1.0 / (1.0 + e2)
        w2_ref[...] = e2 / (1.0 + e2)
        a1_ref[...] = a1
        a2_ref[...] = a2
        out_ref[...] = jnp.zeros_like(out_ref)

    x = x_ref[...]
    acc = jnp.zeros((TOKENS, HIDDEN), jnp.float32)
    for i in range(E_BLK):
        e = g * E_BLK + i
        y = _swiglu(x, wg_ref[i], wu_ref[i], wd_ref[i])
        w = (jnp.where(a1_ref[...] == e, w1_ref[...], 0.0)
             + jnp.where(a2_ref[...] == e, w2_ref[...], 0.0))
        acc = acc + y * w
    out_ref[...] += acc


def _moe_main_kernel(x_ref, ids_ref, vals_ref, prev_ref,
                     wg_ref, wu_ref, wd_ref, out_ref):
    g = pl.program_id(0)

    @pl.when(g == 0)
    def _init():
        out_ref[...] = prev_ref[...]

    a1 = ids_ref[:, 0:1]
    a2 = ids_ref[:, 1:2]
    v1 = vals_ref[:, 0:1]
    v2 = vals_ref[:, 1:2]
    e2 = jnp.exp(v2 - v1)
    denom = 1.0 + e2
    w1 = 1.0 / denom
    w2 = e2 / denom

    x = x_ref[...]
    acc = jnp.zeros((TOKENS, HIDDEN), jnp.float32)
    for i in range(E_BLK):
        e = PRE_EXPERTS + g * E_BLK + i
        y = _swiglu(x, wg_ref[i], wu_ref[i], wd_ref[i])
        w = (jnp.where(a1 == e, w1, 0.0)
             + jnp.where(a2 == e, w2, 0.0))
        acc = acc + y * w
    out_ref[...] += acc


@jax.jit
def kernel(hidden_states, router_logits, W_gate, W_up, W_down):
    topk_ids, topk_vals = _routing_sc(router_logits)

    partial = pl.pallas_call(
        _moe_pre_kernel,
        grid=(PRE_EXPERTS // E_BLK,),
        in_specs=[
            pl.BlockSpec((TOKENS, HIDDEN), lambda g: (0, 0)),
            pl.BlockSpec((TOKENS, NUM_EXPERTS), lambda g: (0, 0)),
            pl.BlockSpec((E_BLK, HIDDEN, D_FF), lambda g: (g, 0, 0)),
            pl.BlockSpec((E_BLK, HIDDEN, D_FF), lambda g: (g, 0, 0)),
            pl.BlockSpec((E_BLK, D_FF, HIDDEN), lambda g: (g, 0, 0)),
        ],
        out_specs=pl.BlockSpec((TOKENS, HIDDEN), lambda g: (0, 0)),
        out_shape=jax.ShapeDtypeStruct((TOKENS, HIDDEN), jnp.float32),
        scratch_shapes=[
            pltpu.VMEM((TOKENS, 1), jnp.float32),
            pltpu.VMEM((TOKENS, 1), jnp.float32),
            pltpu.VMEM((TOKENS, 1), jnp.int32),
            pltpu.VMEM((TOKENS, 1), jnp.int32),
        ],
        compiler_params=pltpu.CompilerParams(
            dimension_semantics=("arbitrary",),
        ),
    )(hidden_states, router_logits, W_gate, W_up, W_down)

    n_pre_blocks = PRE_EXPERTS // E_BLK
    return pl.pallas_call(
        _moe_main_kernel,
        grid=((NUM_EXPERTS - PRE_EXPERTS) // E_BLK,),
        in_specs=[
            pl.BlockSpec((TOKENS, HIDDEN), lambda g: (0, 0)),
            pl.BlockSpec((TOKENS, PAD_COLS), lambda g: (0, 0)),
            pl.BlockSpec((TOKENS, PAD_COLS), lambda g: (0, 0)),
            pl.BlockSpec((TOKENS, HIDDEN), lambda g: (0, 0)),
            pl.BlockSpec((E_BLK, HIDDEN, D_FF),
                         lambda g: (g + n_pre_blocks, 0, 0)),
            pl.BlockSpec((E_BLK, HIDDEN, D_FF),
                         lambda g: (g + n_pre_blocks, 0, 0)),
            pl.BlockSpec((E_BLK, D_FF, HIDDEN),
                         lambda g: (g + n_pre_blocks, 0, 0)),
        ],
        out_specs=pl.BlockSpec((TOKENS, HIDDEN), lambda g: (0, 0)),
        out_shape=jax.ShapeDtypeStruct((TOKENS, HIDDEN), jnp.float32),
        compiler_params=pltpu.CompilerParams(
            dimension_semantics=("arbitrary",),
        ),
    )(hidden_states, topk_ids, topk_vals, partial, W_gate, W_up, W_down)


# speedup vs baseline: 1.0090x; 1.0090x over previous
import functools

import jax
import jax.numpy as jnp
from jax import lax
from jax.experimental import pallas as pl
from jax.experimental.pallas import tpu as pltpu
from jax.experimental.pallas import tpu_sc as plsc

NUM_EXPERTS = 64
TOP_K = 2
HIDDEN = 1024
D_FF = 512
TOKENS = 128

E_BLK = 2
PAD_COLS = 16

_SC_INFO = plsc.get_sparse_core_info()
_NC = _SC_INFO.num_cores
_NS = _SC_INFO.num_subcores
_NW = _NC * _NS
_TPW = TOKENS // _NW


def _routing_sc_body(logits_hbm, ids_hbm, vals_hbm, loc_v, ids_v, vals_v):
    wid = lax.axis_index("s") * _NC + lax.axis_index("c")
    base = wid * _TPW
    pltpu.sync_copy(logits_hbm.at[pl.ds(base, _TPW)], loc_v)
    iota = lax.iota(jnp.int32, 16)
    for j in range(_TPW):
        rows = [loc_v[j, pl.ds(k * 16, 16)] for k in range(NUM_EXPERTS // 16)]
        m1 = jnp.max(rows[0])
        for r in rows[1:]:
            m1 = jnp.maximum(m1, jnp.max(r))
        a1 = jnp.int32(NUM_EXPERTS)
        for k, r in enumerate(rows):
            cand = jnp.min(jnp.where(r == m1, iota + 16 * k, NUM_EXPERTS))
            a1 = jnp.minimum(a1, cand)
        m2 = jnp.float32(-1e30)
        masked = []
        for k, r in enumerate(rows):
            rm = jnp.where(iota + 16 * k == a1, -1e30, r)
            masked.append(rm)
            m2 = jnp.maximum(m2, jnp.max(rm))
        a2 = jnp.int32(NUM_EXPERTS)
        for k, rm in enumerate(masked):
            cand = jnp.min(jnp.where(rm == m2, iota + 16 * k, NUM_EXPERTS))
            a2 = jnp.minimum(a2, cand)
        ids_v[j, pl.ds(0, 16)] = jnp.where(
            iota == 0, a1, jnp.where(iota == 1, a2, 0))
        vals_v[j, pl.ds(0, 16)] = jnp.where(
            iota == 0, m1, jnp.where(iota == 1, m2, 0.0))
    pltpu.sync_copy(ids_v, ids_hbm.at[pl.ds(base, _TPW)])
    pltpu.sync_copy(vals_v, vals_hbm.at[pl.ds(base, _TPW)])


_routing_sc = functools.partial(
    pl.kernel,
    out_type=[
        jax.ShapeDtypeStruct((TOKENS, PAD_COLS), jnp.int32),
        jax.ShapeDtypeStruct((TOKENS, PAD_COLS), jnp.float32),
    ],
    mesh=plsc.VectorSubcoreMesh(core_axis_name="c", subcore_axis_name="s"),
    compiler_params=pltpu.CompilerParams(needs_layout_passes=False),
    scratch_types=[
        pltpu.VMEM((_TPW, NUM_EXPERTS), jnp.float32),
        pltpu.VMEM((_TPW, PAD_COLS), jnp.int32),
        pltpu.VMEM((_TPW, PAD_COLS), jnp.float32),
    ],
)(_routing_sc_body)


def _swiglu(x, wg, wu, wd):
    gate = jnp.dot(x, wg, preferred_element_type=jnp.float32)
    up = jnp.dot(x, wu, preferred_element_type=jnp.float32)
    act = gate * jax.nn.sigmoid(gate) * up
    return jnp.dot(act, wd, preferred_element_type=jnp.float32)


def _moe_main_kernel(x_ref, ids_ref, vals_ref,
                     wg_ref, wu_ref, wd_ref, out_ref):
    g = pl.program_id(0)

    @pl.when(g == 0)
    def _init():
        out_ref[...] = jnp.zeros_like(out_ref)

    a1 = ids_ref[:, 0:1]
    a2 = ids_ref[:, 1:2]
    v1 = vals_ref[:, 0:1]
    v2 = vals_ref[:, 1:2]
    e2 = jnp.exp(v2 - v1)
    denom = 1.0 + e2
    w1 = 1.0 / denom
    w2 = e2 / denom

    x = x_ref[...]
    acc = jnp.zeros((TOKENS, HIDDEN), jnp.float32)
    for i in range(E_BLK):
        e = g * E_BLK + i
        y = _swiglu(x, wg_ref[i], wu_ref[i], wd_ref[i])
        w = (jnp.where(a1 == e, w1, 0.0)
             + jnp.where(a2 == e, w2, 0.0))
        acc = acc + y * w
    out_ref[...] += acc


@jax.jit
def kernel(hidden_states, router_logits, W_gate, W_up, W_down):
    topk_ids, topk_vals = _routing_sc(router_logits)

    return pl.pallas_call(
        _moe_main_kernel,
        grid=(NUM_EXPERTS // E_BLK,),
        in_specs=[
            pl.BlockSpec((TOKENS, HIDDEN), lambda g: (0, 0)),
            pl.BlockSpec((TOKENS, PAD_COLS), lambda g: (0, 0)),
            pl.BlockSpec((TOKENS, PAD_COLS), lambda g: (0, 0)),
            pl.BlockSpec((E_BLK, HIDDEN, D_FF), lambda g: (g, 0, 0)),
            pl.BlockSpec((E_BLK, HIDDEN, D_FF), lambda g: (g, 0, 0)),
            pl.BlockSpec((E_BLK, D_FF, HIDDEN), lambda g: (g, 0, 0)),
        ],
        out_specs=pl.BlockSpec((TOKENS, HIDDEN), lambda g: (0, 0)),
        out_shape=jax.ShapeDtypeStruct((TOKENS, HIDDEN), jnp.float32),
        compiler_params=pltpu.CompilerParams(
            dimension_semantics=("arbitrary",),
        ),
    )(hidden_states, topk_ids, topk_vals, W_gate, W_up, W_down)


# speedup vs baseline: 1.0161x; 1.0071x over previous
import functools

import jax
import jax.numpy as jnp
from jax import lax
from jax.experimental import pallas as pl
from jax.experimental.pallas import tpu as pltpu
from jax.experimental.pallas import tpu_sc as plsc

NUM_EXPERTS = 64
TOP_K = 2
HIDDEN = 1024
D_FF = 512
TOKENS = 128

E_BLK = 2
PAD_COLS = 16

_NC = 1
_NS = 16
_NW = _NC * _NS
_TPW = TOKENS // _NW


def _routing_sc_body(logits_hbm, ids_hbm, vals_hbm, loc_v, ids_v, vals_v):
    wid = lax.axis_index("s") * _NC + lax.axis_index("c")
    base = wid * _TPW
    pltpu.sync_copy(logits_hbm.at[pl.ds(base, _TPW)], loc_v)
    iota = lax.iota(jnp.int32, 16)
    for j in range(_TPW):
        rows = [loc_v[j, pl.ds(k * 16, 16)] for k in range(NUM_EXPERTS // 16)]
        m1 = jnp.max(rows[0])
        for r in rows[1:]:
            m1 = jnp.maximum(m1, jnp.max(r))
        a1 = jnp.int32(NUM_EXPERTS)
        for k, r in enumerate(rows):
            cand = jnp.min(jnp.where(r == m1, iota + 16 * k, NUM_EXPERTS))
            a1 = jnp.minimum(a1, cand)
        m2 = jnp.float32(-1e30)
        masked = []
        for k, r in enumerate(rows):
            rm = jnp.where(iota + 16 * k == a1, -1e30, r)
            masked.append(rm)
            m2 = jnp.maximum(m2, jnp.max(rm))
        a2 = jnp.int32(NUM_EXPERTS)
        for k, rm in enumerate(masked):
            cand = jnp.min(jnp.where(rm == m2, iota + 16 * k, NUM_EXPERTS))
            a2 = jnp.minimum(a2, cand)
        ids_v[j, pl.ds(0, 16)] = jnp.where(
            iota == 0, a1, jnp.where(iota == 1, a2, 0))
        vals_v[j, pl.ds(0, 16)] = jnp.where(
            iota == 0, m1, jnp.where(iota == 1, m2, 0.0))
    pltpu.sync_copy(ids_v, ids_hbm.at[pl.ds(base, _TPW)])
    pltpu.sync_copy(vals_v, vals_hbm.at[pl.ds(base, _TPW)])


_routing_sc = functools.partial(
    pl.kernel,
    out_type=[
        jax.ShapeDtypeStruct((TOKENS, PAD_COLS), jnp.int32),
        jax.ShapeDtypeStruct((TOKENS, PAD_COLS), jnp.float32),
    ],
    mesh=plsc.VectorSubcoreMesh(core_axis_name="c", subcore_axis_name="s",
                                num_cores=_NC),
    compiler_params=pltpu.CompilerParams(needs_layout_passes=False),
    scratch_types=[
        pltpu.VMEM((_TPW, NUM_EXPERTS), jnp.float32),
        pltpu.VMEM((_TPW, PAD_COLS), jnp.int32),
        pltpu.VMEM((_TPW, PAD_COLS), jnp.float32),
    ],
)(_routing_sc_body)


def _swiglu(x, wg, wu, wd):
    gate = jnp.dot(x, wg, preferred_element_type=jnp.float32)
    up = jnp.dot(x, wu, preferred_element_type=jnp.float32)
    act = gate * jax.nn.sigmoid(gate) * up
    return jnp.dot(act, wd, preferred_element_type=jnp.float32)


def _moe_main_kernel(x_ref, ids_ref, vals_ref,
                     wg_ref, wu_ref, wd_ref, out_ref):
    g = pl.program_id(0)

    @pl.when(g == 0)
    def _init():
        out_ref[...] = jnp.zeros_like(out_ref)

    a1 = ids_ref[:, 0:1]
    a2 = ids_ref[:, 1:2]
    v1 = vals_ref[:, 0:1]
    v2 = vals_ref[:, 1:2]
    e2 = jnp.exp(v2 - v1)
    denom = 1.0 + e2
    w1 = 1.0 / denom
    w2 = e2 / denom

    x = x_ref[...]
    acc = jnp.zeros((TOKENS, HIDDEN), jnp.float32)
    for i in range(E_BLK):
        e = g * E_BLK + i
        y = _swiglu(x, wg_ref[i], wu_ref[i], wd_ref[i])
        w = (jnp.where(a1 == e, w1, 0.0)
             + jnp.where(a2 == e, w2, 0.0))
        acc = acc + y * w
    out_ref[...] += acc


@jax.jit
def kernel(hidden_states, router_logits, W_gate, W_up, W_down):
    topk_ids, topk_vals = _routing_sc(router_logits)

    return pl.pallas_call(
        _moe_main_kernel,
        grid=(NUM_EXPERTS // E_BLK,),
        in_specs=[
            pl.BlockSpec((TOKENS, HIDDEN), lambda g: (0, 0)),
            pl.BlockSpec((TOKENS, PAD_COLS), lambda g: (0, 0)),
            pl.BlockSpec((TOKENS, PAD_COLS), lambda g: (0, 0)),
            pl.BlockSpec((E_BLK, HIDDEN, D_FF), lambda g: (g, 0, 0)),
            pl.BlockSpec((E_BLK, HIDDEN, D_FF), lambda g: (g, 0, 0)),
            pl.BlockSpec((E_BLK, D_FF, HIDDEN), lambda g: (g, 0, 0)),
        ],
        out_specs=pl.BlockSpec((TOKENS, HIDDEN), lambda g: (0, 0)),
        out_shape=jax.ShapeDtypeStruct((TOKENS, HIDDEN), jnp.float32),
        compiler_params=pltpu.CompilerParams(
            dimension_semantics=("arbitrary",),
        ),
    )(hidden_states, topk_ids, topk_vals, W_gate, W_up, W_down)
